# Initial kernel scaffold; baseline (speedup 1.0000x reference)
import jax
import jax.numpy as jnp
from jax.experimental import pallas as pl


def kernel(x, edge_index, W1_l, b1_l, W1_r, W2_l, b2_l, W2_r):
    raise NotImplementedError("write your pallas kernel here")


# speedup vs baseline: 5.7941x; 5.7941x over previous
import functools

import jax
import jax.numpy as jnp
from jax import lax
from jax.experimental import pallas as pl
from jax.experimental.pallas import tpu as pltpu
from jax.experimental.pallas import tpu_sc as plsc

N = 10000
D = 128
NC, NS = 2, 16
NW = NC * NS
NPAD = 10240
RZ = NPAD // NS


def _chunking(E):
    per_w = E // NW
    assert per_w * NW == E, E
    for c in range(128, 0, -1):
        if per_w % c == 0:
            return per_w // c, c
    raise AssertionError(E)


def _seg_body(nch, gather, *refs):
    if gather:
        (x_hbm, srcw, dstw, zacc, accp,
         acc_sh, src_v, dst_v, rows_v, sem) = refs
    else:
        (dstw, zacc, ones_hbm, accp,
         acc_sh, dst_v, rows_v, sem) = refs
    c = lax.axis_index("c")
    s = lax.axis_index("s")
    wid = s * NC + c

    pltpu.sync_copy(zacc.at[pl.ds(s * RZ, RZ)], acc_sh.at[pl.ds(s * RZ, RZ)])
    if not gather:
        pltpu.sync_copy(ones_hbm, rows_v)
    plsc.subcore_barrier()

    def chunk(j, carry):
        k = wid * nch + j
        pltpu.sync_copy(dstw.at[k], dst_v)
        if gather:
            pltpu.sync_copy(srcw.at[k], src_v)
            pltpu.async_copy(x_hbm.at[src_v.at[0]], rows_v, sem).wait()
        pltpu.sync_copy(rows_v, acc_sh.at[dst_v.at[0]], add=True)
        return carry

    lax.fori_loop(0, nch, chunk, 0)
    plsc.subcore_barrier()

    pltpu.sync_copy(acc_sh.at[pl.ds(s * RZ, RZ)],
                    accp.at[c, pl.ds(s * RZ, RZ)])


def _make_seg(nch, cs, gather):
    mesh = plsc.VectorSubcoreMesh(core_axis_name="c", subcore_axis_name="s",
                                  num_cores=NC, num_subcores=NS)
    f32 = jnp.float32
    out_type = jax.ShapeDtypeStruct((NC, NPAD, D), f32)
    scratch = [pltpu.VMEM_SHARED((NPAD, D), f32)]
    if gather:
        scratch.append(pltpu.VMEM((1, cs), jnp.int32))
    scratch += [
        pltpu.VMEM((1, cs), jnp.int32),
        pltpu.VMEM((cs, D), f32),
        pltpu.SemaphoreType.DMA,
    ]
    return pl.kernel(functools.partial(_seg_body, nch, gather),
                     out_type=out_type, mesh=mesh, scratch_types=scratch)


def _tc_body(relu, accp, cntp, x, w, b, out_ref):
    a = accp[0] + accp[1]
    cnt = cntp[0, :, 0:1] + cntp[1, :, 0:1]
    mean = a * (1.0 / jnp.maximum(cnt, 1.0))
    cat = jnp.concatenate([mean, x[...]], axis=1)
    o = jnp.dot(cat, w[...], preferred_element_type=jnp.float32) + b[...]
    if relu:
        o = jnp.maximum(o, 0.0)
    out_ref[...] = o


def _tc_layer(accp, cntp, x, wcat, b, relu):
    R = 1000
    grid = (N // R,)
    return pl.pallas_call(
        functools.partial(_tc_body, relu),
        grid=grid,
        in_specs=[
            pl.BlockSpec((NC, R, D), lambda i: (0, i, 0)),
            pl.BlockSpec((NC, R, D), lambda i: (0, i, 0)),
            pl.BlockSpec((R, D), lambda i: (i, 0)),
            pl.BlockSpec((2 * D, D), lambda i: (0, 0)),
            pl.BlockSpec((1, D), lambda i: (0, 0)),
        ],
        out_specs=pl.BlockSpec((R, D), lambda i: (i, 0)),
        out_shape=jax.ShapeDtypeStruct((N, D), jnp.float32),
    )(accp, cntp, x, wcat, b)


def kernel(x, edge_index, W1_l, b1_l, W1_r, W2_l, b2_l, W2_r):
    E = edge_index.shape[1]
    nch, cs = _chunking(E)
    ei = edge_index.astype(jnp.int32)
    srcw = ei[0].reshape(NW * nch, 1, cs)
    dstw = ei[1].reshape(NW * nch, 1, cs)
    f32 = jnp.float32
    zacc = jnp.zeros((NPAD, D), f32)
    ones = jnp.ones((cs, D), f32)

    seg = _make_seg(nch, cs, gather=True)
    cntk = _make_seg(nch, cs, gather=False)

    cntp = cntk(dstw, zacc, ones)
    accp1 = seg(x, srcw, dstw, zacc)
    h = _tc_layer(accp1, cntp, x,
                  jnp.concatenate([W1_l, W1_r], axis=0),
                  b1_l.reshape(1, D), relu=True)
    accp2 = seg(h, srcw, dstw, zacc)
    out = _tc_layer(accp2, cntp, h,
                    jnp.concatenate([W2_l, W2_r], axis=0),
                    b2_l.reshape(1, D), relu=False)
    return out


# speedup vs baseline: 8.8385x; 1.5254x over previous
import functools

import jax
import jax.numpy as jnp
from jax import lax
from jax.experimental import pallas as pl
from jax.experimental.pallas import tpu as pltpu
from jax.experimental.pallas import tpu_sc as plsc

N = 10000
D = 128
NC, NS = 2, 16
NW = NC * NS
NPAD = 10240
RZ = NPAD // NS


def _chunking(E):
    per_w = E // NW
    assert per_w * NW == E, E
    for c in range(128, 0, -1):
        if per_w % c == 0:
            return per_w // c, c
    raise AssertionError(E)


def _seg_body(nch, gather, *refs):
    if gather:
        (x_hbm, srcw, dstw, zacc, accp, acc_sh,
         src0, src1, dst0, dst1, rows0, rows1, sem0, sem1) = refs
        src_v, dst_v = (src0, src1), (dst0, dst1)
        rows_v, sems = (rows0, rows1), (sem0, sem1)
    else:
        (dstw, zacc, ones_hbm, accp,
         acc_sh, dst0, dst1, ones_v, isem0, isem1) = refs
        dst_v, isems = (dst0, dst1), (isem0, isem1)
    c = lax.axis_index("c")
    s = lax.axis_index("s")
    wid = s * NC + c
    base = wid * nch
    npair = nch // 2

    pltpu.sync_copy(zacc.at[pl.ds(s * RZ, RZ)], acc_sh.at[pl.ds(s * RZ, RZ)])
    if not gather:
        pltpu.sync_copy(ones_hbm, ones_v)
    plsc.subcore_barrier()

    if gather:
        for b in (0, 1):
            pltpu.sync_copy(srcw.at[base + b], src_v[b])
            pltpu.sync_copy(dstw.at[base + b], dst_v[b])
            pltpu.async_copy(x_hbm.at[src_v[b].at[0]], rows_v[b], sems[b])

        def pair(p, carry):
            for b in (0, 1):
                j = 2 * p + b
                pltpu.make_async_copy(x_hbm.at[src_v[b].at[0]],
                                      rows_v[b], sems[b]).wait()
                pltpu.sync_copy(rows_v[b], acc_sh.at[dst_v[b].at[0]],
                                add=True)

                @pl.when(p < npair - 1)
                def _():
                    pltpu.sync_copy(srcw.at[base + j + 2], src_v[b])
                    pltpu.sync_copy(dstw.at[base + j + 2], dst_v[b])
                    pltpu.async_copy(x_hbm.at[src_v[b].at[0]],
                                     rows_v[b], sems[b])
            return carry

        lax.fori_loop(0, npair, pair, 0)
    else:
        for b in (0, 1):
            pltpu.async_copy(dstw.at[base + b], dst_v[b], isems[b])

        def pair(p, carry):
            for b in (0, 1):
                j = 2 * p + b
                pltpu.make_async_copy(dstw.at[base + j],
                                      dst_v[b], isems[b]).wait()
                pltpu.sync_copy(ones_v, acc_sh.at[dst_v[b].at[0]], add=True)

                @pl.when(p < npair - 1)
                def _():
                    pltpu.async_copy(dstw.at[base + j + 2], dst_v[b],
                                     isems[b])
            return carry

        lax.fori_loop(0, npair, pair, 0)
    plsc.subcore_barrier()

    pltpu.sync_copy(acc_sh.at[pl.ds(s * RZ, RZ)],
                    accp.at[c, pl.ds(s * RZ, RZ)])


def _make_seg(nch, cs, gather):
    mesh = plsc.VectorSubcoreMesh(core_axis_name="c", subcore_axis_name="s",
                                  num_cores=NC, num_subcores=NS)
    f32 = jnp.float32
    out_type = jax.ShapeDtypeStruct((NC, NPAD, D), f32)
    scratch = [pltpu.VMEM_SHARED((NPAD, D), f32)]
    if gather:
        scratch += [
            pltpu.VMEM((1, cs), jnp.int32), pltpu.VMEM((1, cs), jnp.int32),
            pltpu.VMEM((1, cs), jnp.int32), pltpu.VMEM((1, cs), jnp.int32),
            pltpu.VMEM((cs, D), f32), pltpu.VMEM((cs, D), f32),
            pltpu.SemaphoreType.DMA, pltpu.SemaphoreType.DMA,
        ]
    else:
        scratch += [
            pltpu.VMEM((1, cs), jnp.int32), pltpu.VMEM((1, cs), jnp.int32),
            pltpu.VMEM((cs, D), f32),
            pltpu.SemaphoreType.DMA, pltpu.SemaphoreType.DMA,
        ]
    return pl.kernel(functools.partial(_seg_body, nch, gather),
                     out_type=out_type, mesh=mesh, scratch_types=scratch)


def _tc_body(relu, accp, cntp, x, w, b, out_ref):
    a = accp[0] + accp[1]
    cnt = cntp[0, :, 0:1] + cntp[1, :, 0:1]
    mean = a * (1.0 / jnp.maximum(cnt, 1.0))
    cat = jnp.concatenate([mean, x[...]], axis=1)
    o = jnp.dot(cat, w[...], preferred_element_type=jnp.float32) + b[...]
    if relu:
        o = jnp.maximum(o, 0.0)
    out_ref[...] = o


def _tc_layer(accp, cntp, x, wcat, b, relu):
    R = 1000
    grid = (N // R,)
    return pl.pallas_call(
        functools.partial(_tc_body, relu),
        grid=grid,
        in_specs=[
            pl.BlockSpec((NC, R, D), lambda i: (0, i, 0)),
            pl.BlockSpec((NC, R, D), lambda i: (0, i, 0)),
            pl.BlockSpec((R, D), lambda i: (i, 0)),
            pl.BlockSpec((2 * D, D), lambda i: (0, 0)),
            pl.BlockSpec((1, D), lambda i: (0, 0)),
        ],
        out_specs=pl.BlockSpec((R, D), lambda i: (i, 0)),
        out_shape=jax.ShapeDtypeStruct((N, D), jnp.float32),
    )(accp, cntp, x, wcat, b)


def kernel(x, edge_index, W1_l, b1_l, W1_r, W2_l, b2_l, W2_r):
    E = edge_index.shape[1]
    nch, cs = _chunking(E)
    ei = edge_index.astype(jnp.int32)
    srcw = ei[0].reshape(NW * nch, 1, cs)
    dstw = ei[1].reshape(NW * nch, 1, cs)
    f32 = jnp.float32
    zacc = jnp.zeros((NPAD, D), f32)
    ones = jnp.ones((cs, D), f32)

    seg = _make_seg(nch, cs, gather=True)
    cntk = _make_seg(nch, cs, gather=False)

    cntp = cntk(dstw, zacc, ones)
    accp1 = seg(x, srcw, dstw, zacc)
    h = _tc_layer(accp1, cntp, x,
                  jnp.concatenate([W1_l, W1_r], axis=0),
                  b1_l.reshape(1, D), relu=True)
    accp2 = seg(h, srcw, dstw, zacc)
    out = _tc_layer(accp2, cntp, h,
                    jnp.concatenate([W2_l, W2_r], axis=0),
                    b2_l.reshape(1, D), relu=False)
    return out
